# baseline (speedup 1.0000x reference)
import jax
import jax.numpy as jnp
from jax.experimental import pallas as pl
from jax.experimental.pallas import tpu as pltpu

N = 10000
F = 128
BM = 200
NS = 2


def _hgc_kernel(x_ref, adj0_ref, adj1_ref, w_ref, b_ref, nw_ref, sw_ref,
                out_ref, sup_ref):
    i = pl.program_id(0)

    @pl.when(i == 0)
    def _compute_support():
        x = x_ref[...]
        na = jax.nn.sigmoid(
            jnp.sum(x * nw_ref[...], axis=1, keepdims=True))
        sa = jax.nn.softmax(sw_ref[...], axis=-1)
        xw = x * na * sa
        sup = jax.lax.dot_general(
            xw, w_ref[...], (((1,), (0,)), ((), ())),
            preferred_element_type=jnp.float32)
        sup_ref[...] = sup.astype(jnp.bfloat16)

    sup = sup_ref[...]
    b = b_ref[...]
    for s, adj_ref in enumerate((adj0_ref, adj1_ref)):
        adj_blk = adj_ref[...].astype(jnp.bfloat16)
        acc = jax.lax.dot_general(
            adj_blk, sup, (((1,), (0,)), ((), ())),
            preferred_element_type=jnp.float32)
        out_ref[pl.ds(s * BM, BM), :] = acc + b


@jax.jit
def kernel(x, adj, W, b, node_w, sem_w):
    nw = node_w.reshape(1, F)
    sw = sem_w.reshape(1, F)
    bb = b.reshape(1, F)
    grid = (N // (NS * BM),)
    out = pl.pallas_call(
        _hgc_kernel,
        grid=grid,
        in_specs=[
            pl.BlockSpec((N, F), lambda i: (0, 0)),
            pl.BlockSpec((BM, N), lambda i: (2 * i, 0)),
            pl.BlockSpec((BM, N), lambda i: (2 * i + 1, 0)),
            pl.BlockSpec((F, F), lambda i: (0, 0)),
            pl.BlockSpec((1, F), lambda i: (0, 0)),
            pl.BlockSpec((1, F), lambda i: (0, 0)),
            pl.BlockSpec((1, F), lambda i: (0, 0)),
        ],
        out_specs=pl.BlockSpec((NS * BM, F), lambda i: (i, 0)),
        out_shape=jax.ShapeDtypeStruct((N, F), jnp.float32),
        scratch_shapes=[pltpu.VMEM((N, F), jnp.bfloat16)],
    )(x, adj, adj, W, bb, nw, sw)
    return out


# speedup vs baseline: 1.0293x; 1.0293x over previous
import jax
import jax.numpy as jnp
from jax.experimental import pallas as pl
from jax.experimental.pallas import tpu as pltpu

N = 10000
F = 128
BM = 400


def _hgc_kernel(x_ref, adj_ref, w_ref, b_ref, nw_ref, sw_ref, out_ref,
                sup_ref):
    i = pl.program_id(0)

    @pl.when(i == 0)
    def _compute_support():
        x = x_ref[...]
        na = jax.nn.sigmoid(
            jnp.sum(x * nw_ref[...], axis=1, keepdims=True))
        sa = jax.nn.softmax(sw_ref[...], axis=-1)
        xw = x * na * sa
        sup = jax.lax.dot_general(
            xw, w_ref[...], (((1,), (0,)), ((), ())),
            preferred_element_type=jnp.float32)
        sup_ref[...] = sup.astype(jnp.bfloat16)

    adj_blk = adj_ref[...].astype(jnp.bfloat16)
    acc = jax.lax.dot_general(
        adj_blk, sup_ref[...], (((1,), (0,)), ((), ())),
        preferred_element_type=jnp.float32)
    out_ref[...] = acc + b_ref[...]


@jax.jit
def kernel(x, adj, W, b, node_w, sem_w):
    nw = node_w.reshape(1, F)
    sw = sem_w.reshape(1, F)
    bb = b.reshape(1, F)
    grid = (N // BM,)
    out = pl.pallas_call(
        _hgc_kernel,
        grid=grid,
        in_specs=[
            pl.BlockSpec((N, F), lambda i: (0, 0)),
            pl.BlockSpec((BM, N), lambda i: (i, 0)),
            pl.BlockSpec((F, F), lambda i: (0, 0)),
            pl.BlockSpec((1, F), lambda i: (0, 0)),
            pl.BlockSpec((1, F), lambda i: (0, 0)),
            pl.BlockSpec((1, F), lambda i: (0, 0)),
        ],
        out_specs=pl.BlockSpec((BM, F), lambda i: (i, 0)),
        out_shape=jax.ShapeDtypeStruct((N, F), jnp.float32),
        scratch_shapes=[pltpu.VMEM((N, F), jnp.bfloat16)],
    )(x, adj, W, bb, nw, sw)
    return out
